# Initial kernel scaffold; baseline (speedup 1.0000x reference)
import jax
import jax.numpy as jnp
from jax.experimental import pallas as pl


def kernel(input):
    raise NotImplementedError("write your pallas kernel here")


# speedup vs baseline: 1.5147x; 1.5147x over previous
import jax
import jax.numpy as jnp
from jax import lax
from jax.experimental import pallas as pl
from jax.experimental.pallas import tpu as pltpu
from jax.experimental.pallas import tpu_sc as plsc

SEQ, BATCH, VOCAB = 50, 1024, 1000
ROWS = SEQ * BATCH
NC, NS, L = 2, 16, 16
NW = NC * NS
RPW = ROWS // NW
WORDS_PW = RPW * VOCAB
ZWORDS = 64000
NZ = WORDS_PW // ZWORDS
IDX_W = 128
NIDX = (RPW + IDX_W - 1) // IDX_W


def _body(in_hbm, out_hbm, zbuf, idx_v, offs2d, ones_v, zsem, ssem):
    wid = lax.axis_index("s") * NC + lax.axis_index("c")
    base_row = wid * RPW
    base_word = base_row * VOCAB

    pltpu.sync_copy(in_hbm.at[pl.ds(base_row, RPW)], idx_v)

    zeros16 = jnp.zeros((L,), jnp.float32)

    def zb(i, carry):
        zbuf[pl.ds(i * L, L)] = zeros16
        return carry

    lax.fori_loop(0, ZWORDS // L, zb, None)

    for z in range(NZ):
        pltpu.make_async_copy(
            zbuf, out_hbm.at[pl.ds(base_word + z * ZWORDS, ZWORDS)], zsem
        ).start()

    iota = lax.broadcasted_iota(jnp.int32, (L,), 0)
    ones16 = jnp.ones((L,), jnp.float32)
    for g in range(IDX_W // L):
        ones_v[pl.ds(g * L, L)] = ones16
    for k in range(NIDX * IDX_W // L):
        kk = k if k * L < RPW else k - (NIDX * IDX_W - RPW) // L
        pos = kk * L + iota
        idxv = idx_v[pl.ds(kk * L, L)]
        offs = (base_row + pos) * VOCAB + idxv
        offs2d[k * L // IDX_W, pl.ds(k * L % IDX_W, L)] = offs

    for z in range(NZ):
        pltpu.make_async_copy(
            zbuf, out_hbm.at[pl.ds(base_word + z * ZWORDS, ZWORDS)], zsem
        ).wait()
    for k in range(NIDX):
        pltpu.make_async_copy(ones_v, out_hbm.at[offs2d.at[k]], ssem).start()
    for k in range(NIDX):
        pltpu.make_async_copy(ones_v, out_hbm.at[offs2d.at[k]], ssem).wait()


def _onehot_sc(flat_idx):
    mesh = plsc.VectorSubcoreMesh(core_axis_name="c", subcore_axis_name="s")
    return pl.kernel(
        _body,
        mesh=mesh,
        out_type=jax.ShapeDtypeStruct((ROWS * VOCAB,), jnp.float32),
        scratch_types=[
            pltpu.VMEM((ZWORDS,), jnp.float32),
            pltpu.VMEM((RPW,), jnp.int32),
            pltpu.VMEM((NIDX, IDX_W), jnp.int32),
            pltpu.VMEM((IDX_W,), jnp.float32),
            pltpu.SemaphoreType.DMA,
            pltpu.SemaphoreType.DMA,
        ],
    )(flat_idx)


def kernel(input):
    flat_idx = input.reshape(ROWS).astype(jnp.int32)
    out = _onehot_sc(flat_idx)
    return out.reshape(SEQ, BATCH, VOCAB)


# speedup vs baseline: 2.8315x; 1.8693x over previous
import jax
import jax.numpy as jnp
from jax import lax
from jax.experimental import pallas as pl
from jax.experimental.pallas import tpu as pltpu
from jax.experimental.pallas import tpu_sc as plsc

SEQ, BATCH, VOCAB = 50, 1024, 1000
ROWS = SEQ * BATCH
NC, NS, L = 2, 16, 16
NW = NC * NS
RPW = ROWS // NW
SLAB = 32
NSLAB = RPW // SLAB
SPS = BATCH // SLAB
CBUF = 1024
NB = 3


def _body(in_hbm, out_hbm, idx_v, buf0, buf1, buf2, sem0, sem1, sem2):
    bufs = [buf0, buf1, buf2]
    sems = [sem0, sem1, sem2]
    wid = lax.axis_index("s") * NC + lax.axis_index("c")
    base_row = wid * RPW
    base_slab = wid * NSLAB

    pltpu.sync_copy(in_hbm.at[pl.ds(base_row, RPW)], idx_v)

    zeros16 = jnp.zeros((L,), jnp.float32)
    iota = lax.broadcasted_iota(jnp.int32, (L,), 0)
    ones16 = jnp.ones((L,), jnp.float32)
    ntail = VOCAB % L
    tail0 = VOCAB - ntail

    def zrow(r, carry):
        for c in range(tail0 // L):
            buf0[r, pl.ds(c * L, L)] = zeros16
            buf1[r, pl.ds(c * L, L)] = zeros16
            buf2[r, pl.ds(c * L, L)] = zeros16
        rows = jnp.broadcast_to(r, (L,)).astype(jnp.int32)
        cols = tail0 + jnp.minimum(iota, ntail - 1)
        plsc.store_scatter(buf0, [rows, cols], zeros16)
        plsc.store_scatter(buf1, [rows, cols], zeros16)
        plsc.store_scatter(buf2, [rows, cols], zeros16)
        return carry

    lax.fori_loop(0, SLAB, zrow, None)

    def set_ones(k, buf, val):
        for g in range(SLAB // L):
            rows = g * L + iota
            cols = idx_v[pl.ds(k * SLAB + g * L, L)]
            plsc.store_scatter(buf, [rows, cols], val)

    def copy_slab(k, buf, sem):
        slab = base_slab + k
        s = slab // SPS
        r0 = (slab % SPS) * SLAB
        return pltpu.make_async_copy(
            buf, out_hbm.at[s, pl.ds(r0, SLAB)], sem
        )

    for k in range(NSLAB):
        b = k % NB
        if k >= NB:
            copy_slab(k - NB, bufs[b], sems[b]).wait()
            set_ones(k - NB, bufs[b], zeros16)
        set_ones(k, bufs[b], ones16)
        copy_slab(k, bufs[b], sems[b]).start()
    for k in range(NSLAB - NB, NSLAB):
        b = k % NB
        copy_slab(k, bufs[b], sems[b]).wait()


def _onehot_sc(flat_idx):
    mesh = plsc.VectorSubcoreMesh(core_axis_name="c", subcore_axis_name="s")
    return pl.kernel(
        _body,
        mesh=mesh,
        out_type=jax.ShapeDtypeStruct((SEQ, BATCH, VOCAB), jnp.float32),
        scratch_types=[
            pltpu.VMEM((RPW,), jnp.int32),
            pltpu.VMEM((SLAB, VOCAB), jnp.float32),
            pltpu.VMEM((SLAB, VOCAB), jnp.float32),
            pltpu.VMEM((SLAB, VOCAB), jnp.float32),
            pltpu.SemaphoreType.DMA,
            pltpu.SemaphoreType.DMA,
            pltpu.SemaphoreType.DMA,
        ],
        compiler_params=pltpu.CompilerParams(
            use_tc_tiling_on_sc=True, needs_layout_passes=False
        ),
    )(flat_idx)


def kernel(input):
    flat_idx = input.reshape(ROWS).astype(jnp.int32)
    return _onehot_sc(flat_idx)


# speedup vs baseline: 8.7468x; 3.0891x over previous
import jax
import jax.numpy as jnp
from jax import lax
from jax.experimental import pallas as pl
from jax.experimental.pallas import tpu as pltpu
from jax.experimental.pallas import tpu_sc as plsc

SEQ, BATCH, VOCAB = 50, 1024, 1000
ROWS = SEQ * BATCH
NC, NS, L = 2, 16, 16
NW = NC * NS
BS = 128
SPB = BATCH // BS
NSLABS = SEQ * SPB
JMAX = -(-NSLABS // NW)
NFULL = NSLABS - NW * (JMAX - 1)


def _slab_coords(sigma):
    s = sigma // SPB
    b0 = (sigma % SPB) * BS
    return s, b0


def _body(in_hbm, out_hbm, idx2d, buf, dsem, isem):
    wid = lax.axis_index("s") * NC + lax.axis_index("c")
    has_last = wid < NFULL

    def idx_copy(j):
        sigma = jnp.minimum(wid + NW * j, NSLABS - 1)
        s, b0 = _slab_coords(sigma)
        return pltpu.make_async_copy(
            in_hbm.at[pl.ds(s * BATCH + b0, BS)], idx2d.at[j], isem
        )

    for j in range(JMAX - 1):
        idx_copy(j).start()

    @pl.when(has_last)
    def _():
        idx_copy(JMAX - 1).start()

    zeros16 = jnp.zeros((L,), jnp.float32)

    def zrow(v, carry):
        for c in range(BS // L):
            buf[v, pl.ds(c * L, L)] = zeros16
        return carry

    lax.fori_loop(0, VOCAB, zrow, None)

    for j in range(JMAX - 1):
        idx_copy(j).wait()

    @pl.when(has_last)
    def _():
        idx_copy(JMAX - 1).wait()

    iota = lax.broadcasted_iota(jnp.int32, (L,), 0)
    ones16 = jnp.ones((L,), jnp.float32)

    def scat(j, val):
        for g in range(BS // L):
            rows = idx2d[j, pl.ds(g * L, L)]
            plsc.store_scatter(buf, [rows, g * L + iota], val)

    def slab_copy(j):
        sigma = jnp.minimum(wid + NW * j, NSLABS - 1)
        s, b0 = _slab_coords(sigma)
        return pltpu.make_async_copy(
            buf, out_hbm.at[s, pl.ds(0, VOCAB), pl.ds(b0, BS)], dsem
        )

    for j in range(JMAX):
        def step(j=j):
            if j > 0:
                slab_copy(j - 1).wait()
                scat(j - 1, zeros16)
            scat(j, ones16)
            slab_copy(j).start()

        if j == JMAX - 1:
            pl.when(has_last)(step)
        else:
            step()

    slab_copy(0).wait()


def _onehot_sc(flat_idx):
    mesh = plsc.VectorSubcoreMesh(core_axis_name="c", subcore_axis_name="s")
    return pl.kernel(
        _body,
        mesh=mesh,
        out_type=jax.ShapeDtypeStruct((SEQ, VOCAB, BATCH), jnp.float32),
        scratch_types=[
            pltpu.VMEM((JMAX, BS), jnp.int32),
            pltpu.VMEM((VOCAB, BS), jnp.float32),
            pltpu.SemaphoreType.DMA,
            pltpu.SemaphoreType.DMA,
        ],
        compiler_params=pltpu.CompilerParams(
            use_tc_tiling_on_sc=True, needs_layout_passes=False
        ),
    )(flat_idx)


def kernel(input):
    flat_idx = input.reshape(ROWS).astype(jnp.int32)
    out = _onehot_sc(flat_idx)
    return jnp.swapaxes(out, 1, 2)


# speedup vs baseline: 8.7521x; 1.0006x over previous
import jax
import jax.numpy as jnp
from jax import lax
from jax.experimental import pallas as pl
from jax.experimental.pallas import tpu as pltpu
from jax.experimental.pallas import tpu_sc as plsc

SEQ, BATCH, VOCAB = 50, 1024, 1000
ROWS = SEQ * BATCH
NC, NS, L = 2, 16, 16
NW = NC * NS
BS = 128
SPB = BATCH // BS
NSLABS = SEQ * SPB
JMAX = -(-NSLABS // NW)
NFULL = NSLABS - NW * (JMAX - 1)


def _slab_coords(sigma):
    s = sigma // SPB
    b0 = (sigma % SPB) * BS
    return s, b0


def _body(in_hbm, out_hbm, idx2d, buf, dsem, isem):
    wid = lax.axis_index("s") * NC + lax.axis_index("c")
    has_last = wid < NFULL

    def idx_copy(j):
        sigma = jnp.minimum(wid + NW * j, NSLABS - 1)
        s, b0 = _slab_coords(sigma)
        return pltpu.make_async_copy(
            in_hbm.at[pl.ds(s * BATCH + b0, BS)], idx2d.at[j], isem
        )

    for j in range(JMAX - 1):
        idx_copy(j).start()

    @pl.when(has_last)
    def _():
        idx_copy(JMAX - 1).start()

    zeros16 = jnp.zeros((L,), jnp.float32)

    def ztile(vt, carry):
        v0 = vt * 8
        for r in range(8):
            for c in range(BS // L):
                buf[v0 + r, pl.ds(c * L, L)] = zeros16
        return carry

    lax.fori_loop(0, VOCAB // 8, ztile, None)

    for j in range(JMAX - 1):
        idx_copy(j).wait()

    @pl.when(has_last)
    def _():
        idx_copy(JMAX - 1).wait()

    iota = lax.broadcasted_iota(jnp.int32, (L,), 0)
    ones16 = jnp.ones((L,), jnp.float32)

    def scat(j, val):
        for g in range(BS // L):
            rows = idx2d[j, pl.ds(g * L, L)]
            plsc.store_scatter(buf, [rows, g * L + iota], val)

    def slab_copy(j):
        sigma = jnp.minimum(wid + NW * j, NSLABS - 1)
        s, b0 = _slab_coords(sigma)
        return pltpu.make_async_copy(
            buf, out_hbm.at[s, pl.ds(0, VOCAB), pl.ds(b0, BS)], dsem
        )

    for j in range(JMAX):
        def step(j=j):
            if j > 0:
                slab_copy(j - 1).wait()
                scat(j - 1, zeros16)
            scat(j, ones16)
            slab_copy(j).start()

        if j == JMAX - 1:
            pl.when(has_last)(step)
        else:
            step()

    slab_copy(0).wait()


def _onehot_sc(flat_idx):
    mesh = plsc.VectorSubcoreMesh(core_axis_name="c", subcore_axis_name="s")
    return pl.kernel(
        _body,
        mesh=mesh,
        out_type=jax.ShapeDtypeStruct((SEQ, VOCAB, BATCH), jnp.float32),
        scratch_types=[
            pltpu.VMEM((JMAX, BS), jnp.int32),
            pltpu.VMEM((VOCAB, BS), jnp.float32),
            pltpu.SemaphoreType.DMA,
            pltpu.SemaphoreType.DMA,
        ],
        compiler_params=pltpu.CompilerParams(
            use_tc_tiling_on_sc=True, needs_layout_passes=False
        ),
    )(flat_idx)


def kernel(input):
    flat_idx = input.reshape(ROWS).astype(jnp.int32)
    out = _onehot_sc(flat_idx)
    return jnp.swapaxes(out, 1, 2)
